# Initial kernel scaffold; baseline (speedup 1.0000x reference)
import jax
import jax.numpy as jnp
from jax.experimental import pallas as pl


def kernel(x, fc_w, fc_bias, emb, W1, b1, g1, be1, m1, v1, W2, b2, g2, be2, m2, v2, W3, b3, g3, be3, m3, v3, Wo, bo):
    raise NotImplementedError("write your pallas kernel here")


# speedup vs baseline: 8.3021x; 8.3021x over previous
import functools

import jax
import jax.numpy as jnp
from jax import lax
from jax.experimental import pallas as pl
from jax.experimental.pallas import tpu as pltpu
from jax.experimental.pallas import tpu_sc as plsc

NC = 2
NS = 16
NW = NC * NS


def _sc_gather(emb, fcw, xo, B, F, D):
    bpw = B // NW

    mesh = plsc.VectorSubcoreMesh(
        core_axis_name="c", subcore_axis_name="s", num_cores=NC,
        num_subcores=NS)

    @functools.partial(
        pl.kernel,
        out_type=(
            jax.ShapeDtypeStruct((B, F, D), jnp.float32),
            jax.ShapeDtypeStruct((B, F), jnp.float32),
        ),
        mesh=mesh,
        scratch_types=[
            pltpu.VMEM((bpw, F), jnp.int32),
            pltpu.VMEM((F, D), jnp.float32),
            pltpu.VMEM((bpw, F), jnp.float32),
            pltpu.SemaphoreType.DMA,
            pltpu.SemaphoreType.DMA,
        ],
    )
    def sc_kernel(emb_hbm, fcw_hbm, xo_hbm, e_hbm, fcg_hbm,
                  idx_v, ebuf, fc_all, sem_e, sem_f):
        wid = lax.axis_index("s") * NC + lax.axis_index("c")
        base = wid * bpw
        pltpu.sync_copy(xo_hbm.at[pl.ds(base, bpw), :], idx_v)

        def body(i, carry):
            idx_row = idx_v.at[i]
            cp_e = pltpu.async_copy(emb_hbm.at[idx_row], ebuf, sem_e)
            cp_f = pltpu.async_copy(fcw_hbm.at[idx_row], fc_all.at[i], sem_f)
            cp_e.wait()
            pltpu.sync_copy(ebuf, e_hbm.at[base + i])
            cp_f.wait()
            return carry

        lax.fori_loop(0, bpw, body, 0)
        pltpu.sync_copy(fc_all, fcg_hbm.at[pl.ds(base, bpw), :])

    return sc_kernel(emb, fcw, xo)


def _mlp_body(nk, e_ref, w1_ref, fcg_ref, s1_ref, c1_ref, w2_ref, s2_ref,
              c2_ref, w3_ref, s3_ref, c3_ref, wot_ref, c0_ref, out_ref,
              acc_ref):
    k = pl.program_id(1)

    @pl.when(k == 0)
    def _init():
        acc_ref[...] = jnp.zeros_like(acc_ref)

    acc_ref[...] += jnp.dot(
        e_ref[...].astype(jnp.bfloat16), w1_ref[...],
        preferred_element_type=jnp.float32)

    @pl.when(k == nk - 1)
    def _finish():
        h1 = jnp.maximum(acc_ref[...] * s1_ref[...] + c1_ref[...], 0.0)
        h2 = jnp.dot(h1.astype(jnp.bfloat16), w2_ref[...],
                     preferred_element_type=jnp.float32)
        h2 = jnp.maximum(h2 * s2_ref[...] + c2_ref[...], 0.0)
        h3 = jnp.dot(h2.astype(jnp.bfloat16), w3_ref[...],
                     preferred_element_type=jnp.float32)
        h3 = jnp.maximum(h3 * s3_ref[...] + c3_ref[...], 0.0)
        lin = jnp.sum(fcg_ref[...], axis=1, keepdims=True)
        dot_o = jnp.sum(h3 * wot_ref[...], axis=1, keepdims=True)
        out_ref[...] = dot_o + lin + c0_ref[...]


def kernel(x, fc_w, fc_bias, emb, W1, b1, g1, be1, m1, v1, W2, b2, g2, be2,
           m2, v2, W3, b3, g3, be3, m3, v3, Wo, bo):
    B, F = x.shape
    D = emb.shape[1]
    K = F * D
    N1 = W1.shape[1]
    N2 = W2.shape[1]
    N3 = W3.shape[1]

    offsets = (jnp.arange(F, dtype=x.dtype) * 1000)[None, :]
    xo = x + offsets

    def fold(g, v, b, m, be):
        s = g * lax.rsqrt(v + 1e-5)
        return s[None, :], ((b - m) * s + be)[None, :]

    s1, c1 = fold(g1, v1, b1, m1, be1)
    s2, c2 = fold(g2, v2, b2, m2, be2)
    s3, c3 = fold(g3, v3, b3, m3, be3)

    e3, fcg = _sc_gather(emb, fc_w.reshape(-1), xo, B, F, D)
    e2 = e3.reshape(B, K)

    BT = 1024
    KT = 1280
    nb = B // BT
    nk = K // KT

    w1b = W1.astype(jnp.bfloat16)
    w2b = W2.astype(jnp.bfloat16)
    w3b = W3.astype(jnp.bfloat16)
    wot = Wo.reshape(1, N3)
    c0 = (fc_bias + bo).reshape(1, 1)

    out = pl.pallas_call(
        functools.partial(_mlp_body, nk),
        grid=(nb, nk),
        in_specs=[
            pl.BlockSpec((BT, KT), lambda b, k: (b, k)),
            pl.BlockSpec((KT, N1), lambda b, k: (k, 0)),
            pl.BlockSpec((BT, F), lambda b, k: (b, 0)),
            pl.BlockSpec((1, N1), lambda b, k: (0, 0)),
            pl.BlockSpec((1, N1), lambda b, k: (0, 0)),
            pl.BlockSpec((N1, N2), lambda b, k: (0, 0)),
            pl.BlockSpec((1, N2), lambda b, k: (0, 0)),
            pl.BlockSpec((1, N2), lambda b, k: (0, 0)),
            pl.BlockSpec((N2, N3), lambda b, k: (0, 0)),
            pl.BlockSpec((1, N3), lambda b, k: (0, 0)),
            pl.BlockSpec((1, N3), lambda b, k: (0, 0)),
            pl.BlockSpec((1, N3), lambda b, k: (0, 0)),
            pl.BlockSpec((1, 1), lambda b, k: (0, 0)),
        ],
        out_specs=pl.BlockSpec((BT, 1), lambda b, k: (b, 0)),
        out_shape=jax.ShapeDtypeStruct((B, 1), jnp.float32),
        scratch_shapes=[pltpu.VMEM((BT, N1), jnp.float32)],
        compiler_params=pltpu.CompilerParams(
            dimension_semantics=("arbitrary", "arbitrary")),
    )(e2, w1b, fcg, s1, c1, w2b, s2, c2, w3b, s3, c3, wot, c0)

    return out


# speedup vs baseline: 13.3862x; 1.6124x over previous
import functools

import jax
import jax.numpy as jnp
from jax import lax
from jax.experimental import pallas as pl
from jax.experimental.pallas import tpu as pltpu
from jax.experimental.pallas import tpu_sc as plsc

NC = 2
NS = 16
NW = NC * NS


def _sc_gather(emb, fcv, xoW, B, F, D):
    bpw = B // NW

    mesh = plsc.VectorSubcoreMesh(
        core_axis_name="c", subcore_axis_name="s", num_cores=NC,
        num_subcores=NS)

    @functools.partial(
        pl.kernel,
        out_type=(
            jax.ShapeDtypeStruct((F, B, D), jnp.float32),
            jax.ShapeDtypeStruct((B,), jnp.float32),
        ),
        mesh=mesh,
        scratch_types=[
            pltpu.VMEM((F, bpw), jnp.int32),
            pltpu.VMEM((bpw, D), jnp.float32),
            pltpu.VMEM((bpw, D), jnp.float32),
            pltpu.VMEM((bpw,), jnp.float32),
            pltpu.VMEM((bpw,), jnp.float32),
            pltpu.VMEM((bpw,), jnp.float32),
            pltpu.SemaphoreType.DMA,
            pltpu.SemaphoreType.DMA,
            pltpu.SemaphoreType.DMA,
            pltpu.SemaphoreType.DMA,
            pltpu.SemaphoreType.DMA,
            pltpu.SemaphoreType.DMA,
        ],
    )
    def sc_kernel(emb_hbm, fcv_hbm, xoW_hbm, e_hbm, lin_hbm,
                  idx_v, rows0, rows1, fcb0, fcb1, acc,
                  sg0, sg1, sf0, sf1, sw0, sw1):
        wid = lax.axis_index("s") * NC + lax.axis_index("c")
        base = wid * bpw
        pltpu.sync_copy(xoW_hbm.at[wid], idx_v)

        zeros = jnp.zeros((16,), jnp.float32)
        for c in range(bpw // 16):
            acc[pl.ds(c * 16, 16)] = zeros

        def body(i, carry):
            f0 = 2 * i
            f1 = 2 * i + 1
            g0 = pltpu.async_copy(emb_hbm.at[idx_v.at[f0]], rows0, sg0)
            c0 = pltpu.async_copy(fcv_hbm.at[idx_v.at[f0]], fcb0, sf0)
            g1 = pltpu.async_copy(emb_hbm.at[idx_v.at[f1]], rows1, sg1)
            c1 = pltpu.async_copy(fcv_hbm.at[idx_v.at[f1]], fcb1, sf1)
            g0.wait()
            w0 = pltpu.async_copy(
                rows0, e_hbm.at[f0, pl.ds(base, bpw), :], sw0)
            g1.wait()
            w1 = pltpu.async_copy(
                rows1, e_hbm.at[f1, pl.ds(base, bpw), :], sw1)
            c0.wait()
            c1.wait()
            for c in range(bpw // 16):
                sl = pl.ds(c * 16, 16)
                acc[sl] = acc[sl] + fcb0[sl] + fcb1[sl]
            w0.wait()
            w1.wait()
            return carry

        lax.fori_loop(0, F // 2, body, 0)
        pltpu.sync_copy(acc, lin_hbm.at[pl.ds(base, bpw)])

    return sc_kernel(emb, fcv, xoW)


def _mlp_body(nk, fb, e_ref, w1_ref, lin_ref, s1_ref, c1_ref, w2_ref, s2_ref,
              c2_ref, w3_ref, s3_ref, c3_ref, wot_ref, c0_ref, out_ref,
              acc_ref):
    k = pl.program_id(1)
    d = e_ref.shape[2]

    @pl.when(k == 0)
    def _init():
        acc_ref[...] = jnp.zeros_like(acc_ref)

    w1v = w1_ref[...]
    part = acc_ref[...]
    for j in range(fb):
        part += jnp.dot(
            e_ref[j].astype(jnp.bfloat16), w1v[j * d:(j + 1) * d, :],
            preferred_element_type=jnp.float32)
    acc_ref[...] = part

    @pl.when(k == nk - 1)
    def _finish():
        h1 = jnp.maximum(acc_ref[...] * s1_ref[...] + c1_ref[...], 0.0)
        h2 = jnp.dot(h1.astype(jnp.bfloat16), w2_ref[...],
                     preferred_element_type=jnp.float32)
        h2 = jnp.maximum(h2 * s2_ref[...] + c2_ref[...], 0.0)
        h3 = jnp.dot(h2.astype(jnp.bfloat16), w3_ref[...],
                     preferred_element_type=jnp.float32)
        h3 = jnp.maximum(h3 * s3_ref[...] + c3_ref[...], 0.0)
        dot_o = jnp.sum(h3 * wot_ref[...], axis=1, keepdims=True)
        out_ref[...] = dot_o + lin_ref[...] + c0_ref[...]


def kernel(x, fc_w, fc_bias, emb, W1, b1, g1, be1, m1, v1, W2, b2, g2, be2,
           m2, v2, W3, b3, g3, be3, m3, v3, Wo, bo):
    B, F = x.shape
    D = emb.shape[1]
    N1 = W1.shape[1]
    N2 = W2.shape[1]
    N3 = W3.shape[1]
    bpw = B // NW

    offsets = (jnp.arange(F, dtype=x.dtype) * 1000)[None, :]
    xo = x + offsets
    xoW = xo.T.reshape(F, NW, bpw).transpose(1, 0, 2)

    def fold(g, v, b, m, be):
        s = g * lax.rsqrt(v + 1e-5)
        return s[None, :], ((b - m) * s + be)[None, :]

    s1, c1 = fold(g1, v1, b1, m1, be1)
    s2, c2 = fold(g2, v2, b2, m2, be2)
    s3, c3 = fold(g3, v3, b3, m3, be3)

    eT, lin = _sc_gather(emb, fc_w.reshape(-1), xoW, B, F, D)
    lin2 = lin.reshape(B, 1)

    BT = 1024
    FB = 10
    nb = B // BT
    nk = F // FB

    w1b = W1.astype(jnp.bfloat16)
    w2b = W2.astype(jnp.bfloat16)
    w3b = W3.astype(jnp.bfloat16)
    wot = Wo.reshape(1, N3)
    c0 = (fc_bias + bo).reshape(1, 1)

    out = pl.pallas_call(
        functools.partial(_mlp_body, nk, FB),
        grid=(nb, nk),
        in_specs=[
            pl.BlockSpec((FB, BT, D), lambda b, k: (k, b, 0)),
            pl.BlockSpec((FB * D, N1), lambda b, k: (k, 0)),
            pl.BlockSpec((BT, 1), lambda b, k: (b, 0)),
            pl.BlockSpec((1, N1), lambda b, k: (0, 0)),
            pl.BlockSpec((1, N1), lambda b, k: (0, 0)),
            pl.BlockSpec((N1, N2), lambda b, k: (0, 0)),
            pl.BlockSpec((1, N2), lambda b, k: (0, 0)),
            pl.BlockSpec((1, N2), lambda b, k: (0, 0)),
            pl.BlockSpec((N2, N3), lambda b, k: (0, 0)),
            pl.BlockSpec((1, N3), lambda b, k: (0, 0)),
            pl.BlockSpec((1, N3), lambda b, k: (0, 0)),
            pl.BlockSpec((1, N3), lambda b, k: (0, 0)),
            pl.BlockSpec((1, 1), lambda b, k: (0, 0)),
        ],
        out_specs=pl.BlockSpec((BT, 1), lambda b, k: (b, 0)),
        out_shape=jax.ShapeDtypeStruct((B, 1), jnp.float32),
        scratch_shapes=[pltpu.VMEM((BT, N1), jnp.float32)],
        compiler_params=pltpu.CompilerParams(
            dimension_semantics=("arbitrary", "arbitrary")),
    )(eT, w1b, lin2, s1, c1, w2b, s2, c2, w3b, s3, c3, wot, c0)

    return out
